# Initial kernel scaffold; baseline (speedup 1.0000x reference)
import jax
import jax.numpy as jnp
from jax.experimental import pallas as pl


def kernel(tokens, lengths, emb, EOT_emb, layer2_EOT_emb):
    raise NotImplementedError("write your pallas kernel here")


# speedup vs baseline: 2.1862x; 2.1862x over previous
import functools

import jax
import jax.numpy as jnp
from jax import lax
from jax.experimental import pallas as pl
from jax.experimental.pallas import tpu as pltpu
from jax.experimental.pallas import tpu_sc as plsc

DIM = 512
CODE_SIZE = 16384
CODE_DEPTH = 8
MAX_LEN = 2048
B = 16
VOCAB = CODE_SIZE + 2
SEQ = MAX_LEN
HALF = SEQ // 2
CHUNK = 64
NCHUNK = HALF // CHUNK
LANES = 16
VPR = DIM // LANES


def _sc_body(toks, emb, eot1, eot2, lens,
             out1, out2, mask,
             idx_v, x_v, acc_v, mask_v, len_v, eot_v, sem):
    b = lax.axis_index("s")
    h = lax.axis_index("c")
    r0w = h * HALF

    pltpu.sync_copy(toks.at[b, :, pl.ds(h * NCHUNK, NCHUNK), :], idx_v)

    def _off(j, _):
        for k in range(1, CODE_DEPTH):
            for c in range(CHUNK // LANES):
                sl = pl.ds(c * LANES, LANES)
                idx_v[k, j, sl] = idx_v[k, j, sl] + k * VOCAB
        return 0
    lax.fori_loop(0, NCHUNK, _off, 0)

    pltpu.sync_copy(lens.at[b], len_v)
    iota = lax.iota(jnp.int32, LANES)
    lv = len_v[...]
    l2 = jnp.minimum(lv + 1, MAX_LEN)

    def _mrow(v, _):
        pos = iota + (r0w + v * LANES)
        mask_v[pl.ds(v * LANES, LANES)] = jnp.where(pos < l2, 1, 0)
        return 0
    lax.fori_loop(0, HALF // LANES, _mrow, 0)
    pltpu.sync_copy(mask_v, mask.at[b, pl.ds(r0w, HALF)])

    def _chunk(j, _):
        r0 = r0w + j * CHUNK
        pltpu.async_copy(emb.at[idx_v.at[0, j]], x_v, sem).wait()
        pltpu.sync_copy(x_v, out1.at[b, pl.ds(r0, CHUNK)])
        pltpu.async_copy(emb.at[idx_v.at[1, j]], acc_v, sem).wait()
        for k in range(2, CODE_DEPTH):
            pltpu.async_copy(emb.at[idx_v.at[k, j]], x_v, sem).wait()

            def _row(r, _):
                for c in range(VPR):
                    sl = pl.ds(c * LANES, LANES)
                    acc_v[r, sl] = acc_v[r, sl] + x_v[r, sl]
                return 0
            lax.fori_loop(0, CHUNK, _row, 0)
        pltpu.sync_copy(acc_v, out2.at[b, pl.ds(r0, CHUNK)])
        return 0
    lax.fori_loop(0, NCHUNK, _chunk, 0)

    @pl.when(h == 0)
    def _():
        pltpu.sync_copy(eot1, eot_v)
        pltpu.sync_copy(eot_v, out1.at[b, pl.ds(0, 1)])
        pltpu.sync_copy(eot2, eot_v)
        pltpu.sync_copy(eot_v, out2.at[b, pl.ds(0, 1)])


@jax.jit
def _run(toks, emb_flat, eot1, eot2, lens):
    kfn = pl.kernel(
        _sc_body,
        out_type=(
            jax.ShapeDtypeStruct((B, SEQ, DIM), jnp.float32),
            jax.ShapeDtypeStruct((B, SEQ, DIM), jnp.float32),
            jax.ShapeDtypeStruct((B, SEQ), jnp.int32),
        ),
        mesh=plsc.VectorSubcoreMesh(core_axis_name="c", subcore_axis_name="s"),
        scratch_types=[
            pltpu.VMEM((CODE_DEPTH, NCHUNK, CHUNK), jnp.int32),
            pltpu.VMEM((CHUNK, DIM), jnp.float32),
            pltpu.VMEM((CHUNK, DIM), jnp.float32),
            pltpu.VMEM((HALF,), jnp.int32),
            pltpu.VMEM((LANES,), jnp.int32),
            pltpu.VMEM((1, DIM), jnp.float32),
            pltpu.SemaphoreType.DMA,
        ],
    )
    return kfn(toks, emb_flat, eot1, eot2, lens)


def kernel(tokens, lengths, emb, EOT_emb, layer2_EOT_emb):
    tok = tokens.reshape(B, CODE_DEPTH, MAX_LEN - 1)
    pad = jnp.full((B, CODE_DEPTH, 1), CODE_SIZE + 1, jnp.int32)
    toks = jnp.concatenate([pad, tok], axis=2)
    toks = toks.reshape(B, CODE_DEPTH, 2 * NCHUNK, CHUNK)
    emb_flat = emb.reshape(CODE_DEPTH * VOCAB, DIM)
    lrep = jnp.broadcast_to(lengths[:, None], (B, LANES))
    return _run(toks, emb_flat, EOT_emb, layer2_EOT_emb, lrep)
